# Initial kernel scaffold; baseline (speedup 1.0000x reference)
import jax
import jax.numpy as jnp
from jax.experimental import pallas as pl


def kernel(x, edge_index, W1, W2, w_att, b_att, W_clf, b_clf):
    raise NotImplementedError("write your pallas kernel here")


# speedup vs baseline: 20.4167x; 20.4167x over previous
import functools

import jax
import jax.numpy as jnp
from jax import lax
from jax.experimental import pallas as pl
from jax.experimental.pallas import tpu as pltpu
from jax.experimental.pallas import tpu_sc as plsc

_NC = 2
_NS = 16
_NW = _NC * _NS
_B = 128
_BLK = 1024


def _cdiv(a, b):
  return (a + b - 1) // b


def _make_seg_rows(n_pad, d, nb):
  rows_per_tile = n_pad // _NS
  chunks = rows_per_tile // _B
  mesh = plsc.VectorSubcoreMesh(
      core_axis_name="c", subcore_axis_name="s",
      num_cores=_NC, num_subcores=_NS)

  @functools.partial(
      pl.kernel,
      out_type=jax.ShapeDtypeStruct((_NC * n_pad, d), jnp.float32),
      mesh=mesh,
      scratch_types=[
          pltpu.VMEM((nb, _B), jnp.int32),
          pltpu.VMEM((nb, _B), jnp.int32),
          pltpu.VMEM((_B, d), jnp.float32),
          pltpu.VMEM_SHARED((n_pad, d), jnp.float32),
          pltpu.SemaphoreType.DMA,
      ],
      compiler_params=pltpu.CompilerParams(needs_layout_passes=False),
  )
  def seg_rows(x_hbm, src_hbm, dst_hbm, out_hbm, src_v, dst_v, rows_v,
               agg_sh, sem):
    cid = lax.axis_index("c")
    sid = lax.axis_index("s")
    wid = cid * _NS + sid

    def zrow(i, carry):
      r = i // 8
      off = (i % 8) * 16
      rows_v[r, pl.ds(off, 16)] = jnp.zeros((16,), jnp.float32)
      return carry
    lax.fori_loop(0, _B * 8, zrow, 0)
    for r in range(chunks):
      pltpu.sync_copy(
          rows_v, agg_sh.at[pl.ds(sid * rows_per_tile + r * _B, _B)])
    plsc.subcore_barrier()

    pltpu.sync_copy(src_hbm.at[wid], src_v)
    pltpu.sync_copy(dst_hbm.at[wid], dst_v)

    def body(j, carry):
      pltpu.async_copy(x_hbm.at[src_v.at[j]], rows_v, sem).wait()
      pltpu.sync_copy(rows_v, agg_sh.at[dst_v.at[j]], add=True)
      return carry
    lax.fori_loop(0, nb, body, 0)

    plsc.subcore_barrier()
    for r in range(chunks):
      off = sid * rows_per_tile + r * _B
      pltpu.sync_copy(agg_sh.at[pl.ds(off, _B)],
                      out_hbm.at[pl.ds(cid * n_pad + off, _B)])

  return seg_rows


def _make_seg_scalar(n_pad, nb):
  mesh = plsc.VectorSubcoreMesh(
      core_axis_name="c", subcore_axis_name="s",
      num_cores=_NC, num_subcores=_NS)

  @functools.partial(
      pl.kernel,
      out_type=jax.ShapeDtypeStruct((_NW, n_pad), jnp.float32),
      mesh=mesh,
      scratch_types=[
          pltpu.VMEM((n_pad,), jnp.float32),
          pltpu.VMEM((nb, _B), jnp.int32),
          pltpu.VMEM((nb, _B), jnp.int32),
          pltpu.VMEM((n_pad,), jnp.float32),
      ],
      compiler_params=pltpu.CompilerParams(needs_layout_passes=False),
  )
  def seg_scalar(att_hbm, src_hbm, dst_hbm, out_hbm, att_v, src_v, dst_v,
                 c_v):
    cid = lax.axis_index("c")
    sid = lax.axis_index("s")
    wid = cid * _NS + sid

    pltpu.sync_copy(att_hbm, att_v)
    pltpu.sync_copy(src_hbm.at[wid], src_v)
    pltpu.sync_copy(dst_hbm.at[wid], dst_v)

    def zbody(i, carry):
      c_v[pl.ds(i * 16, 16)] = jnp.zeros((16,), jnp.float32)
      return carry
    lax.fori_loop(0, n_pad // 16, zbody, 0)

    per_row = _B // 16

    def ebody(i, carry):
      j = i // per_row
      k = (i % per_row) * 16
      dvals = dst_v[j, pl.ds(k, 16)]
      svals = src_v[j, pl.ds(k, 16)]
      gathered = plsc.load_gather(att_v, [dvals])
      plsc.addupdate_scatter(c_v, [svals], gathered)
      return carry
    lax.fori_loop(0, nb * per_row, ebody, 0)

    pltpu.sync_copy(c_v, out_hbm.at[wid])

  return seg_scalar


def _mid_body(p_ref, w1_ref, watt_ref, batt_ref, emb_ref, att_ref):
  blk = p_ref[0] + p_ref[1]
  t = jnp.dot(blk, w1_ref[...], preferred_element_type=jnp.float32,
              precision=lax.Precision.HIGHEST)
  embv = jnp.maximum(t, 0.0)
  emb_ref[...] = embv
  logit = jnp.dot(embv, watt_ref[...], preferred_element_type=jnp.float32,
                  precision=lax.Precision.HIGHEST) + batt_ref[0, 0]
  att_ref[...] = jax.nn.sigmoid(logit)


def _make_mid(n_pad, d, h):
  grid = (n_pad // _BLK,)
  return pl.pallas_call(
      _mid_body,
      grid=grid,
      in_specs=[
          pl.BlockSpec((2, _BLK, d), lambda i: (0, i, 0)),
          pl.BlockSpec((d, h), lambda i: (0, 0)),
          pl.BlockSpec((h, 1), lambda i: (0, 0)),
          pl.BlockSpec((1, 1), lambda i: (0, 0)),
      ],
      out_specs=[
          pl.BlockSpec((_BLK, h), lambda i: (i, 0)),
          pl.BlockSpec((_BLK, 1), lambda i: (i, 0)),
      ],
      out_shape=[
          jax.ShapeDtypeStruct((n_pad, h), jnp.float32),
          jax.ShapeDtypeStruct((n_pad, 1), jnp.float32),
      ],
  )


def _make_fin(n, n_pad, h, c):
  def fin_body(emb_ref, att_ref, cpart_ref, w2_ref, wclf_ref, bclf_ref,
               out_ref, acc_ref):
    i = pl.program_id(0)

    @pl.when(i == 0)
    def _init():
      acc_ref[...] = jnp.zeros_like(acc_ref)

    cb = jnp.sum(cpart_ref[...], axis=0)
    w = att_ref[:, 0] * cb
    row = i * _BLK + lax.broadcasted_iota(jnp.int32, (_BLK,), 0)
    w = jnp.where(row < n, w, 0.0)
    v = jnp.dot(w[None, :], emb_ref[...], preferred_element_type=jnp.float32,
                precision=lax.Precision.HIGHEST)
    acc_ref[0:1, :] += v

    @pl.when(i == pl.num_programs(0) - 1)
    def _final():
      hg = acc_ref[0:1, :] / float(n)
      hg2 = jnp.dot(hg, w2_ref[...], preferred_element_type=jnp.float32,
                    precision=lax.Precision.HIGHEST)
      out_ref[...] = jnp.dot(
          hg2, wclf_ref[...], preferred_element_type=jnp.float32,
          precision=lax.Precision.HIGHEST) + bclf_ref[...]

  grid = (n_pad // _BLK,)
  return pl.pallas_call(
      fin_body,
      grid=grid,
      in_specs=[
          pl.BlockSpec((_BLK, h), lambda i: (i, 0)),
          pl.BlockSpec((_BLK, 1), lambda i: (i, 0)),
          pl.BlockSpec((_NW, _BLK), lambda i: (0, i)),
          pl.BlockSpec((h, h), lambda i: (0, 0)),
          pl.BlockSpec((h, c), lambda i: (0, 0)),
          pl.BlockSpec((1, c), lambda i: (0, 0)),
      ],
      out_specs=pl.BlockSpec((1, c), lambda i: (0, 0)),
      out_shape=jax.ShapeDtypeStruct((1, c), jnp.float32),
      scratch_shapes=[pltpu.VMEM((8, h), jnp.float32)],
  )


def kernel(x, edge_index, W1, W2, w_att, b_att, W_clf, b_clf):
  n, d = x.shape
  h = W1.shape[1]
  c = W_clf.shape[1]
  e = edge_index.shape[1]

  n_pad = _cdiv(n + 1, _NS * _B) * _NS * _B
  e_pad = _cdiv(e, _NW * _B) * _NW * _B
  nb = e_pad // (_NW * _B)

  src = edge_index[0]
  dst = edge_index[1]
  pad = jnp.full((e_pad - e,), n, dtype=jnp.int32)
  src_r = jnp.concatenate([src, pad]).reshape(_NW, nb, _B)
  dst_r = jnp.concatenate([dst, pad]).reshape(_NW, nb, _B)
  x_pad = jnp.concatenate(
      [x, jnp.zeros((n_pad - n, d), dtype=jnp.float32)], axis=0)

  p = _make_seg_rows(n_pad, d, nb)(x_pad, src_r, dst_r)
  p3 = p.reshape(_NC, n_pad, d)

  emb, att = _make_mid(n_pad, d, h)(
      p3, W1, w_att, b_att.reshape(1, 1))

  c_part = _make_seg_scalar(n_pad, nb)(att.reshape(n_pad), src_r, dst_r)

  out = _make_fin(n, n_pad, h, c)(
      emb, att, c_part, W2, W_clf, b_clf.reshape(1, c))
  return out.reshape(c)
